# baseline (speedup 1.0000x reference)
import functools

import jax
import jax.numpy as jnp
from jax import lax
from jax.experimental import pallas as pl
from jax.experimental.pallas import tpu as pltpu
from jax.experimental.pallas import tpu_sc as plsc

D = 32
V = 1000000
NC, NS = 2, 16
NW = NC * NS
BB = 4096 // NW
S = 50
NBLK = V // 128
KPW = 245

mesh = plsc.VectorSubcoreMesh(core_axis_name="c", subcore_axis_name="s")

_params = pltpu.CompilerParams(needs_layout_passes=False)


@functools.partial(
    pl.kernel,
    mesh=mesh,
    out_type=jax.ShapeDtypeStruct((V // 4, 128), jnp.float32),
    scratch_types=[
        pltpu.VMEM((2, D, 128), jnp.float32),
        pltpu.VMEM((2, D, 128), jnp.float32),
        pltpu.VMEM((16, 128), jnp.float32),
        pltpu.SemaphoreType.DMA,
        pltpu.SemaphoreType.DMA,
        pltpu.SemaphoreType.DMA,
        pltpu.SemaphoreType.DMA,
    ],
    compiler_params=_params,
)
def _pack(tbt_hbm, tail_hbm, tbl4_hbm, sbuf, obuf, tbuf, ls0, ls1, ws0, ws1):
    wid = lax.axis_index("s") * NC + lax.axis_index("c")
    lsems = [ls0, ls1]
    wsems = [ws0, ws1]
    iota = lax.iota(jnp.int32, 16)

    def col_of(k):
        c = k * NW + wid
        return jnp.where(c < NBLK, c, wid)

    def start_load(k, buf):
        pltpu.async_copy(
            tbt_hbm.at[:, pl.ds(col_of(k) * 128, 128)], sbuf.at[buf],
            lsems[buf])

    def wait_load(buf):
        pltpu.make_async_copy(
            tbt_hbm.at[:, pl.ds(0, 128)], sbuf.at[buf], lsems[buf]).wait()

    def start_write(k, buf):
        pltpu.async_copy(
            obuf.at[buf], tbl4_hbm.at[pl.ds(col_of(k) * 32, 32)],
            wsems[buf])

    def wait_write(buf):
        pltpu.make_async_copy(
            obuf.at[buf], tbl4_hbm.at[pl.ds(0, 32)], wsems[buf]).wait()

    start_load(0, 0)
    start_load(1, 1)

    bases = [(16 * (t % 2) + iota) * 128 + (t // 2) for t in range(8)]

    @pl.loop(0, KPW + 1, step=2)
    def _(k0):
        for buf in range(2):
            k = k0 + buf

            @pl.when(k < KPW)
            def _():
                wait_load(buf)

                @pl.when(k >= 2)
                def _():
                    wait_write(buf)

                for j in range(32):
                    for t in range(8):
                        rows = 16 * (t % 2) + iota
                        cols = (4 * j + t // 2) + iota * 0
                        val = plsc.load_gather(sbuf.at[buf], [rows, cols])
                        obuf[buf, j, pl.ds(16 * t, 16)] = val

                start_write(k, buf)

                @pl.when(k + 2 < KPW)
                def _():
                    start_load(k + 2, buf)

    wait_write(0)
    wait_write(1)

    @pl.when(wid == NW - 1)
    def _():
        pltpu.sync_copy(tail_hbm, tbuf)
        pltpu.sync_copy(tbuf, tbl4_hbm.at[pl.ds(NBLK * 32, 16)])


@functools.partial(
    pl.kernel,
    mesh=mesh,
    out_type=jax.ShapeDtypeStruct((S, D, 4096), jnp.float32),
    scratch_types=[
        pltpu.VMEM((S, BB), jnp.int32),
        pltpu.VMEM((S, BB), jnp.int32),
        pltpu.VMEM((S, BB), jnp.int32),
        pltpu.VMEM((2, BB, 128), jnp.float32),
        pltpu.VMEM((2, D, BB), jnp.float32),
        pltpu.SemaphoreType.DMA,
        pltpu.SemaphoreType.DMA,
        pltpu.SemaphoreType.DMA,
        pltpu.SemaphoreType.DMA,
    ],
    compiler_params=_params,
)
def _emb(idx_hbm, tbl_hbm, out_hbm, idx_v, row_v, sub_v, gbuf, obuf,
         gsem0, gsem1, osem0, osem1):
    wid = lax.axis_index("s") * NC + lax.axis_index("c")
    base = wid * BB
    pltpu.sync_copy(idx_hbm.at[:, pl.ds(base, BB)], idx_v)

    @pl.loop(0, S)
    def _(s):
        for g in range(BB // 16):
            v = idx_v[s, pl.ds(g * 16, 16)]
            row_v[s, pl.ds(g * 16, 16)] = lax.shift_right_logical(v, 2)
            sub_v[s, pl.ds(g * 16, 16)] = lax.shift_left(
                lax.bitwise_and(v, 3), 5)

    gsems = [gsem0, gsem1]
    osems = [osem0, osem1]

    def start_gather(s, buf):
        pltpu.async_copy(tbl_hbm.at[row_v.at[s]], gbuf.at[buf], gsems[buf])

    def wait_gather(buf):
        pltpu.make_async_copy(
            tbl_hbm.at[row_v.at[0]], gbuf.at[buf], gsems[buf]).wait()

    def start_out(s, buf):
        pltpu.async_copy(
            obuf.at[buf], out_hbm.at[s, :, pl.ds(base, BB)], osems[buf])

    def wait_out(buf):
        pltpu.make_async_copy(
            obuf.at[buf], out_hbm.at[0, :, pl.ds(base, BB)],
            osems[buf]).wait()

    start_gather(0, 0)
    start_gather(1, 1)

    iota = lax.iota(jnp.int32, 16)

    @pl.loop(0, S, step=2)
    def _(s0):
        for buf in range(2):
            s = s0 + buf
            wait_gather(buf)

            @pl.when(s >= 2)
            def _():
                wait_out(buf)

            for g in range(BB // 16):
                rows = g * 16 + iota
                cols0 = sub_v[s, pl.ds(g * 16, 16)]
                for d in range(D):
                    val = plsc.load_gather(gbuf.at[buf], [rows, cols0 + d])
                    obuf[buf, d, pl.ds(g * 16, 16)] = val

            start_out(s, buf)

            @pl.when(s + 2 < S)
            def _():
                start_gather(s + 2, buf)

    wait_out(0)
    wait_out(1)


def kernel(indexes, table):
    idx_t = jnp.transpose(indexes.astype(jnp.int32))
    tbl_t = jnp.transpose(table)
    tail = table[NBLK * 128:].reshape(16, 128)
    tbl4 = _pack(tbl_t, tail)
    out = _emb(idx_t, tbl4)
    return jnp.transpose(out, (2, 0, 1))


# speedup vs baseline: 1.4709x; 1.4709x over previous
import functools

import jax
import jax.numpy as jnp
from jax import lax
from jax.experimental import pallas as pl
from jax.experimental.pallas import tpu as pltpu
from jax.experimental.pallas import tpu_sc as plsc

D = 32
V = 1000000
NC, NS = 2, 16
NW = NC * NS
BB = 4096 // NW
S = 50
NBLK = V // 128
KPW = 245

mesh = plsc.VectorSubcoreMesh(core_axis_name="c", subcore_axis_name="s")

_params = pltpu.CompilerParams(needs_layout_passes=False)


@functools.partial(
    pl.kernel,
    mesh=mesh,
    out_type=jax.ShapeDtypeStruct((V // 4, 128), jnp.float32),
    scratch_types=[
        pltpu.VMEM((2, D, 128), jnp.float32),
        pltpu.VMEM((2, D, 128), jnp.float32),
        pltpu.VMEM((16, 128), jnp.float32),
        pltpu.SemaphoreType.DMA,
        pltpu.SemaphoreType.DMA,
        pltpu.SemaphoreType.DMA,
        pltpu.SemaphoreType.DMA,
    ],
    compiler_params=_params,
)
def _pack(tbt_hbm, tail_hbm, tbl4_hbm, sbuf, obuf, tbuf, ls0, ls1, ws0, ws1):
    wid = lax.axis_index("s") * NC + lax.axis_index("c")
    lsems = [ls0, ls1]
    wsems = [ws0, ws1]
    iota = lax.iota(jnp.int32, 16)

    def col_of(k):
        c = k * NW + wid
        return jnp.where(c < NBLK, c, wid)

    def start_load(k, buf):
        pltpu.async_copy(
            tbt_hbm.at[:, pl.ds(col_of(k) * 128, 128)], sbuf.at[buf],
            lsems[buf])

    def wait_load(buf):
        pltpu.make_async_copy(
            tbt_hbm.at[:, pl.ds(0, 128)], sbuf.at[buf], lsems[buf]).wait()

    def start_write(k, buf):
        pltpu.async_copy(
            obuf.at[buf], tbl4_hbm.at[pl.ds(col_of(k) * 32, 32)],
            wsems[buf])

    def wait_write(buf):
        pltpu.make_async_copy(
            obuf.at[buf], tbl4_hbm.at[pl.ds(0, 32)], wsems[buf]).wait()

    start_load(0, 0)
    start_load(1, 1)

    rvecs = [lax.shift_right_logical(16 * u + iota, 2) for u in range(8)]
    cvecs = [lax.shift_left(lax.bitwise_and(16 * u + iota, 3), 5)
             for u in range(8)]

    def load_f(buf, f):
        return [sbuf[buf, f, pl.ds(16 * u, 16)] for u in range(8)]

    def store_f(buf, f, vals):
        for u in range(8):
            plsc.store_scatter(obuf.at[buf], [rvecs[u], cvecs[u] + f],
                               vals[u])

    @pl.loop(0, KPW + 1, step=2)
    def _(k0):
        for buf in range(2):
            k = k0 + buf

            @pl.when(k < KPW)
            def _():
                wait_load(buf)

                @pl.when(k >= 2)
                def _():
                    wait_write(buf)

                vals = load_f(buf, 0)
                for f in range(D):
                    nxt = load_f(buf, f + 1) if f + 1 < D else None
                    store_f(buf, f, vals)
                    vals = nxt

                start_write(k, buf)

                @pl.when(k + 2 < KPW)
                def _():
                    start_load(k + 2, buf)

    wait_write(0)
    wait_write(1)

    @pl.when(wid == NW - 1)
    def _():
        pltpu.sync_copy(tail_hbm, tbuf)
        pltpu.sync_copy(tbuf, tbl4_hbm.at[pl.ds(NBLK * 32, 16)])


@functools.partial(
    pl.kernel,
    mesh=mesh,
    out_type=jax.ShapeDtypeStruct((S, D, 4096), jnp.float32),
    scratch_types=[
        pltpu.VMEM((S, BB), jnp.int32),
        pltpu.VMEM((S, BB), jnp.int32),
        pltpu.VMEM((S, BB), jnp.int32),
        pltpu.VMEM((2, BB, 128), jnp.float32),
        pltpu.VMEM((2, D, BB), jnp.float32),
        pltpu.SemaphoreType.DMA,
        pltpu.SemaphoreType.DMA,
        pltpu.SemaphoreType.DMA,
        pltpu.SemaphoreType.DMA,
    ],
    compiler_params=_params,
)
def _emb(idx_hbm, tbl_hbm, out_hbm, idx_v, row_v, sub_v, gbuf, obuf,
         gsem0, gsem1, osem0, osem1):
    wid = lax.axis_index("s") * NC + lax.axis_index("c")
    base = wid * BB
    pltpu.sync_copy(idx_hbm.at[:, pl.ds(base, BB)], idx_v)

    @pl.loop(0, S)
    def _(s):
        for g in range(BB // 16):
            v = idx_v[s, pl.ds(g * 16, 16)]
            row_v[s, pl.ds(g * 16, 16)] = lax.shift_right_logical(v, 2)
            sub_v[s, pl.ds(g * 16, 16)] = lax.shift_left(
                lax.bitwise_and(v, 3), 5)

    gsems = [gsem0, gsem1]
    osems = [osem0, osem1]

    def start_gather(s, buf):
        pltpu.async_copy(tbl_hbm.at[row_v.at[s]], gbuf.at[buf], gsems[buf])

    def wait_gather(buf):
        pltpu.make_async_copy(
            tbl_hbm.at[row_v.at[0]], gbuf.at[buf], gsems[buf]).wait()

    def start_out(s, buf):
        pltpu.async_copy(
            obuf.at[buf], out_hbm.at[s, :, pl.ds(base, BB)], osems[buf])

    def wait_out(buf):
        pltpu.make_async_copy(
            obuf.at[buf], out_hbm.at[0, :, pl.ds(base, BB)],
            osems[buf]).wait()

    start_gather(0, 0)
    start_gather(1, 1)

    iota = lax.iota(jnp.int32, 16)

    @pl.loop(0, S, step=2)
    def _(s0):
        for buf in range(2):
            s = s0 + buf
            wait_gather(buf)

            @pl.when(s >= 2)
            def _():
                wait_out(buf)

            for g in range(BB // 16):
                rows = g * 16 + iota
                cols0 = sub_v[s, pl.ds(g * 16, 16)]

                def load8(b):
                    return [
                        plsc.load_gather(gbuf.at[buf], [rows, cols0 + 8 * b + d])
                        for d in range(8)
                    ]

                vals = load8(0)
                for b in range(4):
                    nxt = load8(b + 1) if b + 1 < 4 else None
                    for d in range(8):
                        obuf[buf, 8 * b + d, pl.ds(g * 16, 16)] = vals[d]
                    vals = nxt

            start_out(s, buf)

            @pl.when(s + 2 < S)
            def _():
                start_gather(s + 2, buf)

    wait_out(0)
    wait_out(1)


def kernel(indexes, table):
    idx_t = jnp.transpose(indexes.astype(jnp.int32))
    tbl_t = jnp.transpose(table)
    tail = table[NBLK * 128:].reshape(16, 128)
    tbl4 = _pack(tbl_t, tail)
    out = _emb(idx_t, tbl4)
    return jnp.transpose(out, (2, 0, 1))


# speedup vs baseline: 1.5363x; 1.0445x over previous
import functools

import jax
import jax.numpy as jnp
from jax import lax
from jax.experimental import pallas as pl
from jax.experimental.pallas import tpu as pltpu
from jax.experimental.pallas import tpu_sc as plsc

D = 32
V = 1000000
NC, NS = 2, 16
NW = NC * NS
BB = 4096 // NW
S = 50
NBLK = V // 128
KPW = 245

mesh = plsc.VectorSubcoreMesh(core_axis_name="c", subcore_axis_name="s")

_params = pltpu.CompilerParams(needs_layout_passes=False)


@functools.partial(
    pl.kernel,
    mesh=mesh,
    out_type=jax.ShapeDtypeStruct((V // 4, 128), jnp.float32),
    scratch_types=[
        pltpu.VMEM((2, D, 129), jnp.float32),
        pltpu.VMEM((2, D, 128), jnp.float32),
        pltpu.VMEM((16, 128), jnp.float32),
        pltpu.SemaphoreType.DMA,
        pltpu.SemaphoreType.DMA,
        pltpu.SemaphoreType.DMA,
        pltpu.SemaphoreType.DMA,
    ],
    compiler_params=_params,
)
def _pack(tbt_hbm, tail_hbm, tbl4_hbm, sbuf, obuf, tbuf, ls0, ls1, ws0, ws1):
    wid = lax.axis_index("s") * NC + lax.axis_index("c")
    lsems = [ls0, ls1]
    wsems = [ws0, ws1]
    iota = lax.iota(jnp.int32, 16)

    def col_of(k):
        c = k * NW + wid
        return jnp.where(c < NBLK, c, wid)

    def start_load(k, buf):
        pltpu.async_copy(
            tbt_hbm.at[:, pl.ds(col_of(k) * 128, 128)],
            sbuf.at[buf, :, pl.ds(0, 128)], lsems[buf])

    def wait_load(buf):
        pltpu.make_async_copy(
            tbt_hbm.at[:, pl.ds(0, 128)], sbuf.at[buf, :, pl.ds(0, 128)],
            lsems[buf]).wait()

    def start_write(k, buf):
        pltpu.async_copy(
            obuf.at[buf], tbl4_hbm.at[pl.ds(col_of(k) * 32, 32)],
            wsems[buf])

    def wait_write(buf):
        pltpu.make_async_copy(
            obuf.at[buf], tbl4_hbm.at[pl.ds(0, 32)], wsems[buf]).wait()

    start_load(0, 0)
    start_load(1, 1)

    rows01 = [16 * h + iota for h in range(2)]
    zerov = iota * 0

    def load_j(buf, j):
        cols = [zerov + (4 * j + q) for q in range(4)]
        return [
            plsc.load_gather(sbuf.at[buf], [rows01[t % 2], cols[t // 2]])
            for t in range(8)
        ]

    def store_j(buf, j, vals):
        for t in range(8):
            obuf[buf, j, pl.ds(16 * t, 16)] = vals[t]

    @pl.loop(0, KPW + 1, step=2)
    def _(k0):
        for buf in range(2):
            k = k0 + buf

            @pl.when(k < KPW)
            def _():
                wait_load(buf)

                @pl.when(k >= 2)
                def _():
                    wait_write(buf)

                vals = load_j(buf, 0)
                for j in range(32):
                    nxt = load_j(buf, j + 1) if j + 1 < 32 else None
                    store_j(buf, j, vals)
                    vals = nxt

                start_write(k, buf)

                @pl.when(k + 2 < KPW)
                def _():
                    start_load(k + 2, buf)

    wait_write(0)
    wait_write(1)

    @pl.when(wid == NW - 1)
    def _():
        pltpu.sync_copy(tail_hbm, tbuf)
        pltpu.sync_copy(tbuf, tbl4_hbm.at[pl.ds(NBLK * 32, 16)])


@functools.partial(
    pl.kernel,
    mesh=mesh,
    out_type=jax.ShapeDtypeStruct((S, D, 4096), jnp.float32),
    scratch_types=[
        pltpu.VMEM((S, BB), jnp.int32),
        pltpu.VMEM((S, BB), jnp.int32),
        pltpu.VMEM((S, BB), jnp.int32),
        pltpu.VMEM((2, BB, 129), jnp.float32),
        pltpu.VMEM((2, D, BB), jnp.float32),
        pltpu.SemaphoreType.DMA,
        pltpu.SemaphoreType.DMA,
        pltpu.SemaphoreType.DMA,
        pltpu.SemaphoreType.DMA,
    ],
    compiler_params=_params,
)
def _emb(idx_hbm, tbl_hbm, out_hbm, idx_v, row_v, sub_v, gbuf, obuf,
         gsem0, gsem1, osem0, osem1):
    wid = lax.axis_index("s") * NC + lax.axis_index("c")
    base = wid * BB
    pltpu.sync_copy(idx_hbm.at[:, pl.ds(base, BB)], idx_v)

    @pl.loop(0, S)
    def _(s):
        for g in range(BB // 16):
            v = idx_v[s, pl.ds(g * 16, 16)]
            row_v[s, pl.ds(g * 16, 16)] = lax.shift_right_logical(v, 2)
            sub_v[s, pl.ds(g * 16, 16)] = lax.shift_left(
                lax.bitwise_and(v, 3), 5)

    gsems = [gsem0, gsem1]
    osems = [osem0, osem1]

    def start_gather(s, buf):
        pltpu.async_copy(
            tbl_hbm.at[row_v.at[s]], gbuf.at[buf, :, pl.ds(0, 128)],
            gsems[buf])

    def wait_gather(buf):
        pltpu.make_async_copy(
            tbl_hbm.at[row_v.at[0]], gbuf.at[buf, :, pl.ds(0, 128)],
            gsems[buf]).wait()

    def start_out(s, buf):
        pltpu.async_copy(
            obuf.at[buf], out_hbm.at[s, :, pl.ds(base, BB)], osems[buf])

    def wait_out(buf):
        pltpu.make_async_copy(
            obuf.at[buf], out_hbm.at[0, :, pl.ds(base, BB)],
            osems[buf]).wait()

    start_gather(0, 0)
    start_gather(1, 1)

    iota = lax.iota(jnp.int32, 16)

    @pl.loop(0, S, step=2)
    def _(s0):
        for buf in range(2):
            s = s0 + buf
            wait_gather(buf)

            @pl.when(s >= 2)
            def _():
                wait_out(buf)

            for g in range(BB // 16):
                rows = g * 16 + iota
                cols0 = sub_v[s, pl.ds(g * 16, 16)]

                def load8(b):
                    return [
                        plsc.load_gather(gbuf.at[buf], [rows, cols0 + 8 * b + d])
                        for d in range(8)
                    ]

                vals = load8(0)
                for b in range(4):
                    nxt = load8(b + 1) if b + 1 < 4 else None
                    for d in range(8):
                        obuf[buf, 8 * b + d, pl.ds(g * 16, 16)] = vals[d]
                    vals = nxt

            start_out(s, buf)

            @pl.when(s + 2 < S)
            def _():
                start_gather(s + 2, buf)

    wait_out(0)
    wait_out(1)


def kernel(indexes, table):
    idx_t = jnp.transpose(indexes.astype(jnp.int32))
    tbl_t = jnp.transpose(table)
    tail = table[NBLK * 128:].reshape(16, 128)
    tbl4 = _pack(tbl_t, tail)
    out = _emb(idx_t, tbl4)
    return jnp.transpose(out, (2, 0, 1))


# speedup vs baseline: 1.7043x; 1.1093x over previous
import functools

import jax
import jax.numpy as jnp
from jax import lax
from jax.experimental import pallas as pl
from jax.experimental.pallas import tpu as pltpu
from jax.experimental.pallas import tpu_sc as plsc

D = 32
V = 1000000
NC, NS = 2, 16
NW = NC * NS
BB = 4096 // NW
S = 50
NBLK = V // 128
KPW = 245

mesh = plsc.VectorSubcoreMesh(core_axis_name="c", subcore_axis_name="s")

_params = pltpu.CompilerParams(needs_layout_passes=False)


NT = 4
NG = NBLK // NT
KP2 = 62
W_SK = NT * 128 + 1


@functools.partial(
    pl.kernel,
    mesh=mesh,
    out_type=jax.ShapeDtypeStruct((V // 4, 128), jnp.float32),
    scratch_types=[
        pltpu.VMEM((2, 4, 8, W_SK), jnp.float32),
        pltpu.VMEM((2, NT * 32, 128), jnp.float32),
        pltpu.VMEM((16, 128), jnp.float32),
        pltpu.SemaphoreType.DMA,
        pltpu.SemaphoreType.DMA,
        pltpu.SemaphoreType.DMA,
        pltpu.SemaphoreType.DMA,
    ],
    compiler_params=_params,
)
def _pack(tbt_hbm, tail_hbm, tbl4_hbm, sbuf, obuf, tbuf, ls0, ls1, ws0, ws1):
    wid = lax.axis_index("s") * NC + lax.axis_index("c")
    lsems = [ls0, ls1]
    wsems = [ws0, ws1]
    iota = lax.iota(jnp.int32, 16)

    def grp_of(k):
        g = k * NW + wid
        return jnp.where(g < NG, g, wid)

    def start_load(k, buf):
        g = grp_of(k)
        for r in range(4):
            pltpu.async_copy(
                tbt_hbm.at[pl.ds(8 * r, 8), pl.ds(g * (NT * 128), NT * 128)],
                sbuf.at[buf, r, :, pl.ds(0, NT * 128)], lsems[buf])

    def wait_load(buf):
        for r in range(4):
            pltpu.make_async_copy(
                tbt_hbm.at[pl.ds(0, 8), pl.ds(0, NT * 128)],
                sbuf.at[buf, r, :, pl.ds(0, NT * 128)], lsems[buf]).wait()

    def start_write(k, buf):
        pltpu.async_copy(
            obuf.at[buf], tbl4_hbm.at[pl.ds(grp_of(k) * (NT * 32), NT * 32)],
            wsems[buf])

    def wait_write(buf):
        pltpu.make_async_copy(
            obuf.at[buf], tbl4_hbm.at[pl.ds(0, NT * 32)], wsems[buf]).wait()

    start_load(0, 0)
    start_load(1, 1)

    rvecs = [lax.shift_right_logical(16 * h + iota, 3) for h in range(2)]
    svecs = [lax.bitwise_and(16 * h + iota, 7) for h in range(2)]
    zerov = iota * 0

    def load_js(buf, j):
        cols = [zerov + (4 * j + q) for q in range(4)]
        return [
            plsc.load_gather(
                sbuf.at[buf], [rvecs[t % 2], svecs[t % 2], cols[t // 2]])
            for t in range(8)
        ]

    @pl.loop(0, KP2 + 1, step=2)
    def _(k0):
        for buf in range(2):
            k = k0 + buf

            @pl.when(k < KP2)
            def _():
                wait_load(buf)

                @pl.when(k >= 2)
                def _():
                    wait_write(buf)

                @pl.loop(0, NT * 32, step=8)
                def _(j0):
                    allvals = [load_js(buf, j0 + jj) for jj in range(8)]
                    for jj in range(8):
                        for t in range(8):
                            obuf[buf, j0 + jj, pl.ds(16 * t, 16)] = (
                                allvals[jj][t])

                start_write(k, buf)

                @pl.when(k + 2 < KP2)
                def _():
                    start_load(k + 2, buf)

    wait_write(0)
    wait_write(1)

    @pl.when(wid == NW - 1)
    def _():
        pltpu.sync_copy(tail_hbm, tbuf)
        pltpu.sync_copy(tbuf, tbl4_hbm.at[pl.ds(NBLK * 32, 16)])


@functools.partial(
    pl.kernel,
    mesh=mesh,
    out_type=jax.ShapeDtypeStruct((S, D, 4096), jnp.float32),
    scratch_types=[
        pltpu.VMEM((S, BB), jnp.int32),
        pltpu.VMEM((S, BB), jnp.int32),
        pltpu.VMEM((S, BB), jnp.int32),
        pltpu.VMEM((2, BB, 128), jnp.float32),
        pltpu.VMEM((2, D, BB), jnp.float32),
        pltpu.SemaphoreType.DMA,
        pltpu.SemaphoreType.DMA,
        pltpu.SemaphoreType.DMA,
        pltpu.SemaphoreType.DMA,
    ],
    compiler_params=_params,
)
def _emb(idx_hbm, tbl_hbm, out_hbm, idx_v, row_v, sub_v, gbuf, obuf,
         gsem0, gsem1, osem0, osem1):
    wid = lax.axis_index("s") * NC + lax.axis_index("c")
    base = wid * BB
    pltpu.sync_copy(idx_hbm.at[:, pl.ds(base, BB)], idx_v)

    @pl.loop(0, S)
    def _(s):
        for g in range(BB // 16):
            v = idx_v[s, pl.ds(g * 16, 16)]
            row_v[s, pl.ds(g * 16, 16)] = lax.shift_right_logical(v, 2)
            sub_v[s, pl.ds(g * 16, 16)] = lax.shift_left(
                lax.bitwise_and(v, 3), 5)

    gsems = [gsem0, gsem1]
    osems = [osem0, osem1]

    def start_gather(s, buf):
        pltpu.async_copy(tbl_hbm.at[row_v.at[s]], gbuf.at[buf], gsems[buf])

    def wait_gather(buf):
        pltpu.make_async_copy(
            tbl_hbm.at[row_v.at[0]], gbuf.at[buf], gsems[buf]).wait()

    def start_out(s, buf):
        pltpu.async_copy(
            obuf.at[buf], out_hbm.at[s, :, pl.ds(base, BB)], osems[buf])

    def wait_out(buf):
        pltpu.make_async_copy(
            obuf.at[buf], out_hbm.at[0, :, pl.ds(base, BB)],
            osems[buf]).wait()

    start_gather(0, 0)
    start_gather(1, 1)

    iota = lax.iota(jnp.int32, 16)

    @pl.loop(0, S, step=2)
    def _(s0):
        for buf in range(2):
            s = s0 + buf
            wait_gather(buf)

            @pl.when(s >= 2)
            def _():
                wait_out(buf)

            for g in range(BB // 16):
                rows = g * 16 + iota
                cols0 = sub_v[s, pl.ds(g * 16, 16)]

                def load8(b):
                    return [
                        plsc.load_gather(gbuf.at[buf], [rows, cols0 + 8 * b + d])
                        for d in range(8)
                    ]

                vals = load8(0)
                for b in range(4):
                    nxt = load8(b + 1) if b + 1 < 4 else None
                    for d in range(8):
                        obuf[buf, 8 * b + d, pl.ds(g * 16, 16)] = vals[d]
                    vals = nxt

            start_out(s, buf)

            @pl.when(s + 2 < S)
            def _():
                start_gather(s + 2, buf)

    wait_out(0)
    wait_out(1)


def kernel(indexes, table):
    idx_t = jnp.transpose(indexes.astype(jnp.int32))
    tbl_t = jnp.transpose(table)
    tail = table[NBLK * 128:].reshape(16, 128)
    tbl4 = _pack(tbl_t, tail)
    out = _emb(idx_t, tbl4)
    return jnp.transpose(out, (2, 0, 1))


# speedup vs baseline: 1.7237x; 1.0114x over previous
import functools

import jax
import jax.numpy as jnp
from jax import lax
from jax.experimental import pallas as pl
from jax.experimental.pallas import tpu as pltpu
from jax.experimental.pallas import tpu_sc as plsc

D = 32
V = 1000000
NC, NS = 2, 16
NW = NC * NS
BB = 4096 // NW
S = 50
NBLK = V // 128
KPW = 245

mesh = plsc.VectorSubcoreMesh(core_axis_name="c", subcore_axis_name="s")

_params = pltpu.CompilerParams(needs_layout_passes=False)


NT = 4
NG = NBLK // NT
KP2 = 62
W_SK = NT * 128 + 1


@functools.partial(
    pl.kernel,
    mesh=mesh,
    out_type=jax.ShapeDtypeStruct((V // 4, 128), jnp.float32),
    scratch_types=[
        pltpu.VMEM((2, D, 513), jnp.float32),
        pltpu.VMEM((2, NT * 32, 129), jnp.float32),
        pltpu.VMEM((16, 128), jnp.float32),
        pltpu.SemaphoreType.DMA,
        pltpu.SemaphoreType.DMA,
        pltpu.SemaphoreType.DMA,
        pltpu.SemaphoreType.DMA,
    ],
    compiler_params=_params,
)
def _pack(tbt_hbm, tail_hbm, tbl4_hbm, sbuf, obuf, tbuf, ls0, ls1, ws0, ws1):
    wid = lax.axis_index("s") * NC + lax.axis_index("c")
    lsems = [ls0, ls1]
    wsems = [ws0, ws1]
    iota = lax.iota(jnp.int32, 16)

    def grp_of(k):
        g = k * NW + wid
        return jnp.where(g < NG, g, wid)

    def start_load(k, buf):
        g = grp_of(k)
        for r in range(4):
            pltpu.async_copy(
                tbt_hbm.at[pl.ds(8 * r, 8), pl.ds(g * (NT * 128), NT * 128)],
                sbuf.at[buf, pl.ds(8 * r, 8), pl.ds(0, NT * 128)],
                lsems[buf])

    def wait_load(buf):
        for r in range(4):
            pltpu.make_async_copy(
                tbt_hbm.at[pl.ds(0, 8), pl.ds(0, NT * 128)],
                sbuf.at[buf, pl.ds(8 * r, 8), pl.ds(0, NT * 128)],
                lsems[buf]).wait()

    def start_write(k, buf):
        pltpu.async_copy(
            obuf.at[buf, :, pl.ds(0, 128)],
            tbl4_hbm.at[pl.ds(grp_of(k) * (NT * 32), NT * 32)],
            wsems[buf])

    def wait_write(buf):
        pltpu.make_async_copy(
            obuf.at[buf, :, pl.ds(0, 128)],
            tbl4_hbm.at[pl.ds(0, NT * 32)], wsems[buf]).wait()

    start_load(0, 0)
    start_load(1, 1)


    @pl.loop(0, KP2 + 1, step=2)
    def _(k0):
        for buf in range(2):
            k = k0 + buf

            @pl.when(k < KP2)
            def _():
                wait_load(buf)

                @pl.when(k >= 2)
                def _():
                    wait_write(buf)

                @pl.loop(0, NT * 128 // 16)
                def _(u):
                    lvec = 16 * u + iota
                    jv = lax.shift_right_logical(lvec, 2)
                    cv = lax.shift_left(lax.bitwise_and(lvec, 3), 5)
                    vals = [sbuf[buf, f, pl.ds(16 * u, 16)]
                            for f in range(D)]
                    for f in range(D):
                        plsc.store_scatter(
                            obuf.at[buf], [jv, cv + f], vals[f])

                start_write(k, buf)

                @pl.when(k + 2 < KP2)
                def _():
                    start_load(k + 2, buf)

    wait_write(0)
    wait_write(1)

    @pl.when(wid == NW - 1)
    def _():
        pltpu.sync_copy(tail_hbm, tbuf)
        pltpu.sync_copy(tbuf, tbl4_hbm.at[pl.ds(NBLK * 32, 16)])


@functools.partial(
    pl.kernel,
    mesh=mesh,
    out_type=jax.ShapeDtypeStruct((S, D, 4096), jnp.float32),
    scratch_types=[
        pltpu.VMEM((S, BB), jnp.int32),
        pltpu.VMEM((S, BB), jnp.int32),
        pltpu.VMEM((S, BB), jnp.int32),
        pltpu.VMEM((2, BB, 128), jnp.float32),
        pltpu.VMEM((2, D, BB), jnp.float32),
        pltpu.SemaphoreType.DMA,
        pltpu.SemaphoreType.DMA,
        pltpu.SemaphoreType.DMA,
        pltpu.SemaphoreType.DMA,
    ],
    compiler_params=_params,
)
def _emb(idx_hbm, tbl_hbm, out_hbm, idx_v, row_v, sub_v, gbuf, obuf,
         gsem0, gsem1, osem0, osem1):
    wid = lax.axis_index("s") * NC + lax.axis_index("c")
    base = wid * BB
    pltpu.sync_copy(idx_hbm.at[:, pl.ds(base, BB)], idx_v)

    @pl.loop(0, S)
    def _(s):
        for g in range(BB // 16):
            v = idx_v[s, pl.ds(g * 16, 16)]
            row_v[s, pl.ds(g * 16, 16)] = lax.shift_right_logical(v, 2)
            sub_v[s, pl.ds(g * 16, 16)] = lax.shift_left(
                lax.bitwise_and(v, 3), 5)

    gsems = [gsem0, gsem1]
    osems = [osem0, osem1]

    def start_gather(s, buf):
        pltpu.async_copy(tbl_hbm.at[row_v.at[s]], gbuf.at[buf], gsems[buf])

    def wait_gather(buf):
        pltpu.make_async_copy(
            tbl_hbm.at[row_v.at[0]], gbuf.at[buf], gsems[buf]).wait()

    def start_out(s, buf):
        pltpu.async_copy(
            obuf.at[buf], out_hbm.at[s, :, pl.ds(base, BB)], osems[buf])

    def wait_out(buf):
        pltpu.make_async_copy(
            obuf.at[buf], out_hbm.at[0, :, pl.ds(base, BB)],
            osems[buf]).wait()

    start_gather(0, 0)
    start_gather(1, 1)

    iota = lax.iota(jnp.int32, 16)

    @pl.loop(0, S, step=2)
    def _(s0):
        for buf in range(2):
            s = s0 + buf
            wait_gather(buf)

            @pl.when(s >= 2)
            def _():
                wait_out(buf)

            for g in range(BB // 16):
                rows = g * 16 + iota
                cols0 = sub_v[s, pl.ds(g * 16, 16)]

                def load8(b):
                    return [
                        plsc.load_gather(gbuf.at[buf], [rows, cols0 + 8 * b + d])
                        for d in range(8)
                    ]

                vals = load8(0)
                for b in range(4):
                    nxt = load8(b + 1) if b + 1 < 4 else None
                    for d in range(8):
                        obuf[buf, 8 * b + d, pl.ds(g * 16, 16)] = vals[d]
                    vals = nxt

            start_out(s, buf)

            @pl.when(s + 2 < S)
            def _():
                start_gather(s + 2, buf)

    wait_out(0)
    wait_out(1)


def kernel(indexes, table):
    idx_t = jnp.transpose(indexes.astype(jnp.int32))
    tbl_t = jnp.transpose(table)
    tail = table[NBLK * 128:].reshape(16, 128)
    tbl4 = _pack(tbl_t, tail)
    out = _emb(idx_t, tbl4)
    return jnp.transpose(out, (2, 0, 1))
